# baseline (speedup 1.0000x reference)
import jax
import jax.numpy as jnp
from jax.experimental import pallas as pl
from jax.experimental.pallas import tpu as pltpu

_BLK = 2048
_SLOTS = 3
_PW = 32


def _copy(x_hbm, xbuf, sems, step, slot):
    return pltpu.make_async_copy(
        x_hbm.at[pl.ds(step * _BLK, _BLK), :],
        xbuf.at[pl.ds(slot * _BLK, _BLK), :],
        sems.at[slot],
    )


def _router_block(x_hbm, w_ref, packed_ref, xbuf, sems):
    i = pl.program_id(0)
    n = pl.num_programs(0)
    slot = jax.lax.rem(i, _SLOTS)

    @pl.when(i == 0)
    def _():
        for s in range(_SLOTS - 1):
            _copy(x_hbm, xbuf, sems, s, s).start()

    pre = i + _SLOTS - 1

    @pl.when(pre < n)
    def _():
        _copy(x_hbm, xbuf, sems, pre, jax.lax.rem(pre, _SLOTS)).start()

    _copy(x_hbm, xbuf, sems, i, slot).wait()

    x = xbuf[pl.ds(slot * _BLK, _BLK), :]
    w = w_ref[...]
    logits = jax.lax.dot_general(
        x, w,
        dimension_numbers=(((1,), (0,)), ((), ())),
        preferred_element_type=jnp.float32,
    )

    m = jnp.max(logits, axis=-1, keepdims=True)
    e = jnp.exp(logits - m)
    z = jnp.sum(e, axis=-1, keepdims=True)
    scores = e / z

    lane = jax.lax.broadcasted_iota(jnp.int32, scores.shape, 1)
    big = jnp.int32(1 << 30)
    s1 = jnp.max(scores, axis=-1)
    i1 = jnp.min(jnp.where(scores == s1[:, None], lane, big), axis=-1)
    masked = jnp.where(lane == i1[:, None], -jnp.inf, scores)
    s2 = jnp.max(masked, axis=-1)
    i2 = jnp.min(jnp.where(masked == s2[:, None], lane, big), axis=-1)

    tot = s1 + s2
    w1 = s1 / tot
    w2 = s2 / tot

    E = logits.shape[-1]
    pad = jnp.zeros((logits.shape[0], _PW - E - 4), jnp.float32)
    packed_ref[...] = jnp.concatenate(
        [logits, w1[:, None], w2[:, None],
         i1[:, None].astype(jnp.float32), i2[:, None].astype(jnp.float32),
         pad], axis=1)


def kernel(hidden_states, W_gate):
    T, H = hidden_states.shape
    E = W_gate.shape[0]
    grid = (T // _BLK,)
    Wt = W_gate.T

    packed = pl.pallas_call(
        _router_block,
        grid=grid,
        in_specs=[
            pl.BlockSpec(memory_space=pltpu.MemorySpace.HBM),
            pl.BlockSpec((H, E), lambda i: (0, 0)),
        ],
        out_specs=pl.BlockSpec((_BLK, _PW), lambda i: (i, 0)),
        out_shape=jax.ShapeDtypeStruct((T, _PW), jnp.float32),
        scratch_shapes=[
            pltpu.VMEM((_SLOTS * _BLK, H), jnp.float32),
            pltpu.SemaphoreType.DMA((_SLOTS,)),
        ],
        compiler_params=pltpu.CompilerParams(
            dimension_semantics=("arbitrary",),
        ),
    )(hidden_states, Wt)

    logits = packed[:, :E]
    tw = packed[:, E:E + 2]
    ti = packed[:, E + 2:E + 4].astype(jnp.int32)
    return (tw, ti, logits)


# speedup vs baseline: 1.0435x; 1.0435x over previous
import jax
import jax.numpy as jnp
from jax.experimental import pallas as pl
from jax.experimental.pallas import tpu as pltpu

_BLK = 2048
_SLOTS = 3
_PW = 128


def _copy(x_hbm, xbuf, sems, step, slot):
    return pltpu.make_async_copy(
        x_hbm.at[pl.ds(step * _BLK, _BLK), :],
        xbuf.at[pl.ds(slot * _BLK, _BLK), :],
        sems.at[slot],
    )


def _router_block(x_hbm, w_ref, packed_ref, xbuf, sems):
    i = pl.program_id(0)
    n = pl.num_programs(0)
    slot = jax.lax.rem(i, _SLOTS)

    @pl.when(i == 0)
    def _():
        for s in range(_SLOTS - 1):
            _copy(x_hbm, xbuf, sems, s, s).start()

    pre = i + _SLOTS - 1

    @pl.when(pre < n)
    def _():
        _copy(x_hbm, xbuf, sems, pre, jax.lax.rem(pre, _SLOTS)).start()

    _copy(x_hbm, xbuf, sems, i, slot).wait()

    x = xbuf[pl.ds(slot * _BLK, _BLK), :]
    w = w_ref[...]
    logits = jax.lax.dot_general(
        x, w,
        dimension_numbers=(((1,), (1,)), ((), ())),
        preferred_element_type=jnp.float32,
    )

    m = jnp.max(logits, axis=-1, keepdims=True)
    e = jnp.exp(logits - m)
    z = jnp.sum(e, axis=-1, keepdims=True)
    scores = e / z

    lane = jax.lax.broadcasted_iota(jnp.int32, scores.shape, 1)
    big = jnp.int32(1 << 30)
    s1 = jnp.max(scores, axis=-1)
    i1 = jnp.min(jnp.where(scores == s1[:, None], lane, big), axis=-1)
    masked = jnp.where(lane == i1[:, None], -jnp.inf, scores)
    s2 = jnp.max(masked, axis=-1)
    i2 = jnp.min(jnp.where(masked == s2[:, None], lane, big), axis=-1)

    tot = s1 + s2
    w1 = s1 / tot
    w2 = s2 / tot

    E = logits.shape[-1]
    pad = jnp.zeros((logits.shape[0], _PW - E - 4), jnp.float32)
    packed_ref[...] = jnp.concatenate(
        [logits, w1[:, None], w2[:, None],
         i1[:, None].astype(jnp.float32), i2[:, None].astype(jnp.float32),
         pad], axis=1)


def kernel(hidden_states, W_gate):
    T, H = hidden_states.shape
    E = W_gate.shape[0]
    grid = (T // _BLK,)

    packed = pl.pallas_call(
        _router_block,
        grid=grid,
        in_specs=[
            pl.BlockSpec(memory_space=pltpu.MemorySpace.HBM),
            pl.BlockSpec((E, H), lambda i: (0, 0)),
        ],
        out_specs=pl.BlockSpec((_BLK, _PW), lambda i: (i, 0)),
        out_shape=jax.ShapeDtypeStruct((T, _PW), jnp.float32),
        scratch_shapes=[
            pltpu.VMEM((_SLOTS * _BLK, H), jnp.float32),
            pltpu.SemaphoreType.DMA((_SLOTS,)),
        ],
        compiler_params=pltpu.CompilerParams(
            dimension_semantics=("arbitrary",),
        ),
    )(hidden_states, W_gate)

    logits = packed[:, :E]
    tw = packed[:, E:E + 2]
    ti = packed[:, E + 2:E + 4].astype(jnp.int32)
    return (tw, ti, logits)
